# Initial kernel scaffold; baseline (speedup 1.0000x reference)
import jax
import jax.numpy as jnp
from jax.experimental import pallas as pl


def kernel(x, edge_index, a_Wl, a_Wr, a_bc, a_W1, a_b1, a_W2, a_b2, a_W3, a_b3, c_Wl, c_Wr, c_bc, c_W1, c_b1, c_W2, c_b2, c_W3, c_b3):
    raise NotImplementedError("write your pallas kernel here")


# speedup vs baseline: 1.0261x; 1.0261x over previous
import functools
import math

import jax
import jax.numpy as jnp
from jax import lax
from jax.experimental import pallas as pl
from jax.experimental.pallas import tpu as pltpu

N = 10000
D = 256
E = 160000
OUT = 10
JITTER = 1e-3

BLK = 1000
GRID = N // BLK


def _mean_body(x_ref, out_ref):
    i = pl.program_id(0)

    @pl.when(i == 0)
    def _():
        out_ref[...] = jnp.zeros_like(out_ref)

    out_ref[...] += jnp.sum(x_ref[...], axis=0, keepdims=True) * (1.0 / N)


def _col_mean(x):
    return pl.pallas_call(
        _mean_body,
        grid=(GRID,),
        in_specs=[pl.BlockSpec((BLK, D), lambda i: (i, 0))],
        out_specs=pl.BlockSpec((1, D), lambda i: (0, 0)),
        out_shape=jax.ShapeDtypeStruct((1, D), jnp.float32),
    )(x)


def _dot_t(a, b):
    return lax.dot_general(a, b, (((1,), (1,)), ((), ())),
                           preferred_element_type=jnp.float32)


def _softplus(v):
    return jnp.maximum(v, 0.0) + jnp.log1p(jnp.exp(-jnp.abs(v)))


def _main_body(x_ref, ss_ref, cnt_ref, mean_ref,
               aWl_ref, aWr_ref, abc_ref, aW1_ref, ab1_ref, aW2_ref, ab2_ref,
               aW3_ref, ab3_ref,
               cWl_ref, cWr_ref, cbc_ref, cW1_ref, cb1_ref, cW2_ref, cb2_ref,
               cW3_ref, cb3_ref,
               conc_ref, val_ref, hc_acc):
    i = pl.program_id(0)
    mean = mean_ref[...]
    xc = x_ref[...] - mean
    cnt = cnt_ref[...]
    inv = 1.0 / jnp.maximum(cnt, 1.0)
    aggc = (ss_ref[...] - cnt * mean) * inv

    za = _dot_t(aggc, aWl_ref[...]) + _dot_t(xc, aWr_ref[...]) + abc_ref[...]
    h = jnp.maximum(za, 0.0) + xc
    h1 = jnp.maximum(_dot_t(h, aW1_ref[...]) + ab1_ref[...], 0.0)
    h2 = jnp.maximum(_dot_t(h1, aW2_ref[...]) + ab2_ref[...], 0.0)
    ao = _dot_t(h2, aW3_ref[...]) + ab3_ref[...]
    conc_ref[...] = _softplus(ao) + JITTER

    zc = _dot_t(aggc, cWl_ref[...]) + _dot_t(xc, cWr_ref[...]) + cbc_ref[...]
    hc_part = jnp.sum(jnp.maximum(zc, 0.0) + xc, axis=0, keepdims=True)

    @pl.when(i == 0)
    def _():
        hc_acc[...] = jnp.zeros_like(hc_acc)

    hc_acc[...] += hc_part

    @pl.when(i == GRID - 1)
    def _():
        v = hc_acc[...]
        v1 = jnp.maximum(_dot_t(v, cW1_ref[...]) + cb1_ref[...], 0.0)
        v2 = jnp.maximum(_dot_t(v1, cW2_ref[...]) + cb2_ref[...], 0.0)
        val_ref[...] = _dot_t(v2, cW3_ref[...]) + cb3_ref[...]


def _dense(x, segsum, cnt2d, mean,
           a_Wl, a_Wr, a_bc, a_W1, a_b1, a_W2, a_b2, a_W3, a_b3,
           c_Wl, c_Wr, c_bc, c_W1, c_b1, c_W2, c_b2, c_W3, c_b3):
    row = lambda i: (i, 0)
    fix = lambda i: (0, 0)
    full = lambda a: pl.BlockSpec(a.shape, fix)
    return pl.pallas_call(
        _main_body,
        grid=(GRID,),
        in_specs=[
            pl.BlockSpec((BLK, D), row),
            pl.BlockSpec((BLK, D), row),
            pl.BlockSpec((BLK, 1), row),
            pl.BlockSpec((1, D), fix),
            full(a_Wl), full(a_Wr), full(a_bc), full(a_W1), full(a_b1),
            full(a_W2), full(a_b2), full(a_W3), full(a_b3),
            full(c_Wl), full(c_Wr), full(c_bc), full(c_W1), full(c_b1),
            full(c_W2), full(c_b2), full(c_W3), full(c_b3),
        ],
        out_specs=[
            pl.BlockSpec((BLK, OUT), row),
            pl.BlockSpec((1, OUT), fix),
        ],
        out_shape=[
            jax.ShapeDtypeStruct((N, OUT), jnp.float32),
            jax.ShapeDtypeStruct((1, OUT), jnp.float32),
        ],
        scratch_shapes=[pltpu.VMEM((1, D), jnp.float32)],
    )(x, segsum, cnt2d, mean,
      a_Wl, a_Wr, a_bc, a_W1, a_b1, a_W2, a_b2, a_W3, a_b3,
      c_Wl, c_Wr, c_bc, c_W1, c_b1, c_W2, c_b2, c_W3, c_b3)


def _segsum_jnp(x, edge_index):
    src = edge_index[0]
    dst = edge_index[1]
    msg = jnp.take(x, src, axis=0)
    ss = jax.ops.segment_sum(msg, dst, num_segments=N)
    cnt = jax.ops.segment_sum(jnp.ones((E,), jnp.float32), dst, num_segments=N)
    return ss, cnt


def kernel(x, edge_index, a_Wl, a_Wr, a_bc, a_W1, a_b1, a_W2, a_b2, a_W3,
           a_b3, c_Wl, c_Wr, c_bc, c_W1, c_b1, c_W2, c_b2, c_W3, c_b3):
    segsum, cnt = _segsum_jnp(x, edge_index)
    mean = _col_mean(x)
    conc, val = _dense(
        x, segsum, cnt.reshape(N, 1), mean,
        a_Wl, a_Wr, a_bc.reshape(1, -1), a_W1, a_b1.reshape(1, -1),
        a_W2, a_b2.reshape(1, -1), a_W3, a_b3.reshape(1, -1),
        c_Wl, c_Wr, c_bc.reshape(1, -1), c_W1, c_b1.reshape(1, -1),
        c_W2, c_b2.reshape(1, -1), c_W3, c_b3.reshape(1, -1))
    return conc.reshape(-1), val.reshape(-1)
